# baseline (speedup 1.0000x reference)
import jax
import jax.numpy as jnp
from jax import lax
from jax.experimental import pallas as pl
from jax.experimental.pallas import tpu as pltpu
from jax.experimental.pallas import tpu_sc as plsc

B, S, D, DFF, E, K = 1, 2048, 768, 3072, 8, 2
T = B * S
BTM = 256
NB = (T * K) // BTM + E
NPAD = NB * BTM

NC, NS = 2, 16
NW = NC * NS
CH = T // NW

BT = 256
NT = T // BT


def _csum0(a, n):
    k = 1
    while k < n:
        z = jnp.zeros((k,) + a.shape[1:], a.dtype)
        a = a + jnp.concatenate([z, a[:n - k]], axis=0)
        k *= 2
    return a


def _csum1(a, n):
    k = 1
    while k < n:
        z = jnp.zeros(a.shape[:1] + (k,), a.dtype)
        a = a + jnp.concatenate([z, a[:, :n - k]], axis=1)
        k *= 2
    return a


def _route_body(x_ref, Wg_ref, bg_ref, p0_ref, p1_ref, w0_ref, w1_ref,
                sb_ref, nb_ref):
    logits = jnp.dot(x_ref[...], Wg_ref[...],
                     preferred_element_type=jnp.float32) + bg_ref[...]
    p = jax.nn.softmax(logits, axis=-1)
    ids = lax.broadcasted_iota(jnp.int32, p.shape, 1)
    v1 = jnp.max(p, axis=-1, keepdims=True)
    i1 = jnp.min(jnp.where(p >= v1, ids, E), axis=-1, keepdims=True)
    m1 = ids == i1
    pm = jnp.where(m1, -jnp.inf, p)
    v2 = jnp.max(pm, axis=-1, keepdims=True)
    i2 = jnp.min(jnp.where(pm >= v2, ids, E), axis=-1, keepdims=True)
    m2 = ids == i2
    denom = v1 + v2
    w0_ref[...] = v1 / denom
    w1_ref[...] = v2 / denom

    oh0 = m1.astype(jnp.int32)
    oh1 = m2.astype(jnp.int32)
    c0 = _csum0(oh0, T)
    c1 = _csum0(oh1, T)
    tot0 = c0[T - 1:T, :]
    tot1 = c1[T - 1:T, :]
    tote = tot0 + tot1
    pb = (tote + BTM - 1) // BTM
    sblk = _csum1(pb, E) - pb
    spad = sblk * BTM
    p0_ref[...] = jnp.sum(oh0 * (spad + c0 - 1), axis=1, keepdims=True)
    p1_ref[...] = jnp.sum(oh1 * (spad + tot0 + c1 - 1), axis=1, keepdims=True)
    sb_ref[...] = sblk
    nb_ref[...] = pb


def _route(xf, Wg, bg):
    return pl.pallas_call(
        _route_body,
        in_specs=[pl.BlockSpec((T, D), lambda: (0, 0)),
                  pl.BlockSpec((D, E), lambda: (0, 0)),
                  pl.BlockSpec((1, E), lambda: (0, 0))],
        out_specs=[pl.BlockSpec((T, 1), lambda: (0, 0)),
                   pl.BlockSpec((T, 1), lambda: (0, 0)),
                   pl.BlockSpec((T, 1), lambda: (0, 0)),
                   pl.BlockSpec((T, 1), lambda: (0, 0)),
                   pl.BlockSpec((1, E), lambda: (0, 0)),
                   pl.BlockSpec((1, E), lambda: (0, 0))],
        out_shape=[jax.ShapeDtypeStruct((T, 1), jnp.int32),
                   jax.ShapeDtypeStruct((T, 1), jnp.int32),
                   jax.ShapeDtypeStruct((T, 1), jnp.float32),
                   jax.ShapeDtypeStruct((T, 1), jnp.float32),
                   jax.ShapeDtypeStruct((1, E), jnp.int32),
                   jax.ShapeDtypeStruct((1, E), jnp.int32)],
    )(xf, Wg, bg.reshape(1, E))


def _sc_scatter_body(x_hbm, p0_hbm, p1_hbm, xs_hbm, idx0_v, idx1_v, rows_v,
                     sem0, sem1, semr):
    wid = lax.axis_index("s") * NC + lax.axis_index("c")
    base = wid * CH
    cp0 = pltpu.async_copy(p0_hbm.at[pl.ds(base, CH)], idx0_v, sem0)
    cp1 = pltpu.async_copy(p1_hbm.at[pl.ds(base, CH)], idx1_v, sem1)
    cpr = pltpu.async_copy(x_hbm.at[pl.ds(base, CH), :], rows_v, semr)
    cp0.wait()
    cp1.wait()
    cpr.wait()
    s0 = pltpu.async_copy(rows_v, xs_hbm.at[idx0_v], sem0)
    s1 = pltpu.async_copy(rows_v, xs_hbm.at[idx1_v], sem1)
    s0.wait()
    s1.wait()


def _sc_scatter(xf, p0, p1):
    mesh = plsc.VectorSubcoreMesh(core_axis_name="c", subcore_axis_name="s",
                                  num_cores=NC, num_subcores=NS)
    return pl.kernel(
        _sc_scatter_body,
        out_type=jax.ShapeDtypeStruct((NPAD, D), jnp.float32),
        mesh=mesh,
        scratch_types=[pltpu.VMEM((CH,), jnp.int32),
                       pltpu.VMEM((CH,), jnp.int32),
                       pltpu.VMEM((CH, D), jnp.float32),
                       pltpu.SemaphoreType.DMA,
                       pltpu.SemaphoreType.DMA,
                       pltpu.SemaphoreType.DMA],
    )(xf, p0, p1)


F = 2
DF = DFF // F


def _gmm_body(sb_ref, nb_ref, xs_ref, W1_ref, b1_ref, W2_ref, b2_ref, ys_ref):
    e = pl.program_id(0)
    f = pl.program_id(1)
    base = sb_ref[e] * BTM

    def step(j, carry):
        off = pl.multiple_of(base + j * BTM, BTM)
        xb = xs_ref[pl.ds(off, BTM), :]
        h = jnp.maximum(
            jnp.dot(xb, W1_ref[0], preferred_element_type=jnp.float32)
            + b1_ref[0], 0.0)
        y = jnp.dot(h, W2_ref[0], preferred_element_type=jnp.float32)
        ys_ref[pl.ds(off, BTM), :] = jnp.where(
            f == 0, y + b2_ref[0], ys_ref[pl.ds(off, BTM), :] + y)
        return carry

    lax.fori_loop(0, nb_ref[e], step, 0)


def _gmm(sb, nb, xs, W1, b1, W2, b2):
    return pl.pallas_call(
        _gmm_body,
        grid_spec=pltpu.PrefetchScalarGridSpec(
            num_scalar_prefetch=2,
            grid=(E, F),
            in_specs=[
                pl.BlockSpec((NPAD, D), lambda e, f, sb, nb: (0, 0)),
                pl.BlockSpec((1, D, DF), lambda e, f, sb, nb: (e, 0, f)),
                pl.BlockSpec((1, 1, DF), lambda e, f, sb, nb: (e, 0, f)),
                pl.BlockSpec((1, DF, D), lambda e, f, sb, nb: (e, f, 0)),
                pl.BlockSpec((1, 1, D), lambda e, f, sb, nb: (e, 0, 0)),
            ],
            out_specs=pl.BlockSpec((NPAD, D), lambda e, f, sb, nb: (0, 0)),
        ),
        out_shape=jax.ShapeDtypeStruct((NPAD, D), jnp.float32),
    )(sb, nb, xs, W1, b1.reshape(E, 1, DFF), W2, b2.reshape(E, 1, D))


def _sc_gather_body(ys_hbm, p0_hbm, p1_hbm, yk0_hbm, yk1_hbm,
                    idx0_v, idx1_v, buf0_v, buf1_v, sem0, sem1):
    wid = lax.axis_index("s") * NC + lax.axis_index("c")
    base = wid * CH
    ci0 = pltpu.async_copy(p0_hbm.at[pl.ds(base, CH)], idx0_v, sem0)
    ci1 = pltpu.async_copy(p1_hbm.at[pl.ds(base, CH)], idx1_v, sem1)
    ci0.wait()
    ci1.wait()
    cp0 = pltpu.async_copy(ys_hbm.at[idx0_v], buf0_v, sem0)
    cp1 = pltpu.async_copy(ys_hbm.at[idx1_v], buf1_v, sem1)
    cp0.wait()
    co0 = pltpu.async_copy(buf0_v, yk0_hbm.at[pl.ds(base, CH), :], sem0)
    cp1.wait()
    co1 = pltpu.async_copy(buf1_v, yk1_hbm.at[pl.ds(base, CH), :], sem1)
    co0.wait()
    co1.wait()


def _sc_gather(ys, p0, p1):
    mesh = plsc.VectorSubcoreMesh(core_axis_name="c", subcore_axis_name="s",
                                  num_cores=NC, num_subcores=NS)
    return pl.kernel(
        _sc_gather_body,
        out_type=[jax.ShapeDtypeStruct((T, D), jnp.float32),
                  jax.ShapeDtypeStruct((T, D), jnp.float32)],
        mesh=mesh,
        scratch_types=[pltpu.VMEM((CH,), jnp.int32),
                       pltpu.VMEM((CH,), jnp.int32),
                       pltpu.VMEM((CH, D), jnp.float32),
                       pltpu.VMEM((CH, D), jnp.float32),
                       pltpu.SemaphoreType.DMA,
                       pltpu.SemaphoreType.DMA],
    )(ys, p0, p1)


def _shared_body(x_ref, Ws1_ref, bs1_ref, Ws2_ref, bs2_ref, scale_ref, sh_ref):
    xb = x_ref[...]
    hs = jnp.maximum(
        jnp.dot(xb, Ws1_ref[...], preferred_element_type=jnp.float32)
        + bs1_ref[...], 0.0)
    sh = jnp.dot(hs, Ws2_ref[...], preferred_element_type=jnp.float32) + bs2_ref[...]
    sh_ref[...] = xb + (xb + sh) * scale_ref[0, 0]


def _shared(xf, Ws1, bs1, Ws2, bs2, scale):
    return pl.pallas_call(
        _shared_body,
        grid=(NT,),
        in_specs=[pl.BlockSpec((BT, D), lambda t: (t, 0)),
                  pl.BlockSpec((D, DFF), lambda t: (0, 0)),
                  pl.BlockSpec((1, DFF), lambda t: (0, 0)),
                  pl.BlockSpec((DFF, D), lambda t: (0, 0)),
                  pl.BlockSpec((1, D), lambda t: (0, 0)),
                  pl.BlockSpec((1, 1), lambda t: (0, 0))],
        out_specs=pl.BlockSpec((BT, D), lambda t: (t, 0)),
        out_shape=jax.ShapeDtypeStruct((T, D), jnp.float32),
    )(xf, Ws1, bs1.reshape(1, DFF), Ws2, bs2.reshape(1, D), scale.reshape(1, 1))


def _combine_body(sh_ref, yk0_ref, yk1_ref, w0_ref, w1_ref, out_ref):
    out_ref[...] = (sh_ref[...] + w0_ref[...] * yk0_ref[...]
                    + w1_ref[...] * yk1_ref[...])


def _combine(sh, yk0, yk1, w0, w1):
    return pl.pallas_call(
        _combine_body,
        grid=(NT,),
        in_specs=[pl.BlockSpec((BT, D), lambda t: (t, 0)),
                  pl.BlockSpec((BT, D), lambda t: (t, 0)),
                  pl.BlockSpec((BT, D), lambda t: (t, 0)),
                  pl.BlockSpec((BT, 1), lambda t: (t, 0)),
                  pl.BlockSpec((BT, 1), lambda t: (t, 0))],
        out_specs=pl.BlockSpec((BT, D), lambda t: (t, 0)),
        out_shape=jax.ShapeDtypeStruct((T, D), jnp.float32),
    )(sh, yk0, yk1, w0, w1)


def kernel(x, Wg, bg, W1, b1, W2, b2, Ws1, bs1, Ws2, bs2, scale):
    xf = x.reshape(T, D)
    p0, p1, w0, w1, sb, nb = _route(xf, Wg, bg)
    p0f, p1f = p0.reshape(T), p1.reshape(T)
    xs = _sc_scatter(xf, p0f, p1f)
    sh = _shared(xf, Ws1, bs1, Ws2, bs2, scale)
    ys = _gmm(sb.reshape(E), nb.reshape(E), xs, W1, b1, W2, b2)
    yk0, yk1 = _sc_gather(ys, p0f, p1f)
    out = _combine(sh, yk0, yk1, w0, w1)
    return out.reshape(x.shape)


# speedup vs baseline: 1.0468x; 1.0468x over previous
import jax
import jax.numpy as jnp
from jax import lax
from jax.experimental import pallas as pl
from jax.experimental.pallas import tpu as pltpu
from jax.experimental.pallas import tpu_sc as plsc

B, S, D, DFF, E, K = 1, 2048, 768, 3072, 8, 2
T = B * S
BTM = 256
NB = (T * K) // BTM + E
NPAD = NB * BTM

NC, NS = 2, 16
NW = NC * NS
CH = T // NW

BT = 256
NT = T // BT


def _csum0(a, n):
    k = 1
    while k < n:
        z = jnp.zeros((k,) + a.shape[1:], a.dtype)
        a = a + jnp.concatenate([z, a[:n - k]], axis=0)
        k *= 2
    return a


def _csum1(a, n):
    k = 1
    while k < n:
        z = jnp.zeros(a.shape[:1] + (k,), a.dtype)
        a = a + jnp.concatenate([z, a[:, :n - k]], axis=1)
        k *= 2
    return a


def _route_body(x_ref, Wg_ref, bg_ref, p0_ref, p1_ref, w0_ref, w1_ref,
                sb_ref, nb_ref):
    logits = jnp.dot(x_ref[...], Wg_ref[...],
                     preferred_element_type=jnp.float32) + bg_ref[...]
    p = jax.nn.softmax(logits, axis=-1)
    ids = lax.broadcasted_iota(jnp.int32, p.shape, 1)
    v1 = jnp.max(p, axis=-1, keepdims=True)
    i1 = jnp.min(jnp.where(p >= v1, ids, E), axis=-1, keepdims=True)
    m1 = ids == i1
    pm = jnp.where(m1, -jnp.inf, p)
    v2 = jnp.max(pm, axis=-1, keepdims=True)
    i2 = jnp.min(jnp.where(pm >= v2, ids, E), axis=-1, keepdims=True)
    m2 = ids == i2
    denom = v1 + v2
    w0_ref[...] = v1 / denom
    w1_ref[...] = v2 / denom

    oh0 = m1.astype(jnp.int32)
    oh1 = m2.astype(jnp.int32)
    c0 = _csum0(oh0, T)
    c1 = _csum0(oh1, T)
    tot0 = c0[T - 1:T, :]
    tot1 = c1[T - 1:T, :]
    tote = tot0 + tot1
    pb = (tote + BTM - 1) // BTM
    sblk = _csum1(pb, E) - pb
    spad = sblk * BTM
    p0_ref[...] = jnp.sum(oh0 * (spad + c0 - 1), axis=1, keepdims=True)
    p1_ref[...] = jnp.sum(oh1 * (spad + tot0 + c1 - 1), axis=1, keepdims=True)
    sb_ref[...] = sblk
    nb_ref[...] = pb


def _route(xf, Wg, bg):
    return pl.pallas_call(
        _route_body,
        in_specs=[pl.BlockSpec((T, D), lambda: (0, 0)),
                  pl.BlockSpec((D, E), lambda: (0, 0)),
                  pl.BlockSpec((1, E), lambda: (0, 0))],
        out_specs=[pl.BlockSpec((T, 1), lambda: (0, 0)),
                   pl.BlockSpec((T, 1), lambda: (0, 0)),
                   pl.BlockSpec((T, 1), lambda: (0, 0)),
                   pl.BlockSpec((T, 1), lambda: (0, 0)),
                   pl.BlockSpec((1, E), lambda: (0, 0)),
                   pl.BlockSpec((1, E), lambda: (0, 0))],
        out_shape=[jax.ShapeDtypeStruct((T, 1), jnp.int32),
                   jax.ShapeDtypeStruct((T, 1), jnp.int32),
                   jax.ShapeDtypeStruct((T, 1), jnp.float32),
                   jax.ShapeDtypeStruct((T, 1), jnp.float32),
                   jax.ShapeDtypeStruct((1, E), jnp.int32),
                   jax.ShapeDtypeStruct((1, E), jnp.int32)],
    )(xf, Wg, bg.reshape(1, E))


def _sc_scatter_body(x_hbm, p0_hbm, p1_hbm, xs_hbm, idx0_v, idx1_v, rows_v,
                     sem0, sem1, semr):
    wid = lax.axis_index("s") * NC + lax.axis_index("c")
    base = wid * CH
    cp0 = pltpu.async_copy(p0_hbm.at[pl.ds(base, CH)], idx0_v, sem0)
    cp1 = pltpu.async_copy(p1_hbm.at[pl.ds(base, CH)], idx1_v, sem1)
    cpr = pltpu.async_copy(x_hbm.at[pl.ds(base, CH), :], rows_v, semr)
    cp0.wait()
    cp1.wait()
    cpr.wait()
    s0 = pltpu.async_copy(rows_v, xs_hbm.at[idx0_v], sem0)
    s1 = pltpu.async_copy(rows_v, xs_hbm.at[idx1_v], sem1)
    s0.wait()
    s1.wait()


def _sc_scatter(xf, p0, p1):
    mesh = plsc.VectorSubcoreMesh(core_axis_name="c", subcore_axis_name="s",
                                  num_cores=NC, num_subcores=NS)
    return pl.kernel(
        _sc_scatter_body,
        out_type=jax.ShapeDtypeStruct((NPAD, D), jnp.float32),
        mesh=mesh,
        scratch_types=[pltpu.VMEM((CH,), jnp.int32),
                       pltpu.VMEM((CH,), jnp.int32),
                       pltpu.VMEM((CH, D), jnp.float32),
                       pltpu.SemaphoreType.DMA,
                       pltpu.SemaphoreType.DMA,
                       pltpu.SemaphoreType.DMA],
    )(xf, p0, p1)


F = 2
DF = DFF // F


def _gmm_body(sb_ref, nb_ref, xs_ref, W1_ref, b1_ref, W2_ref, b2_ref, ys_ref):
    e = pl.program_id(0)
    f = pl.program_id(1)
    base = sb_ref[e] * BTM

    def step(j, carry):
        off = pl.multiple_of(base + j * BTM, BTM)
        xb = xs_ref[pl.ds(off, BTM), :]
        h = jnp.maximum(
            jnp.dot(xb, W1_ref[0], preferred_element_type=jnp.float32)
            + b1_ref[0], 0.0)
        y = jnp.dot(h, W2_ref[0], preferred_element_type=jnp.float32)
        ys_ref[pl.ds(off, BTM), :] = jnp.where(
            f == 0, y + b2_ref[0], ys_ref[pl.ds(off, BTM), :] + y)
        return carry

    lax.fori_loop(0, nb_ref[e], step, 0)


def _gmm(sb, nb, xs, W1, b1, W2, b2):
    return pl.pallas_call(
        _gmm_body,
        grid_spec=pltpu.PrefetchScalarGridSpec(
            num_scalar_prefetch=2,
            grid=(E, F),
            in_specs=[
                pl.BlockSpec((NPAD, D), lambda e, f, sb, nb: (0, 0)),
                pl.BlockSpec((1, D, DF), lambda e, f, sb, nb: (e, 0, f)),
                pl.BlockSpec((1, 1, DF), lambda e, f, sb, nb: (e, 0, f)),
                pl.BlockSpec((1, DF, D), lambda e, f, sb, nb: (e, f, 0)),
                pl.BlockSpec((1, 1, D), lambda e, f, sb, nb: (e, 0, 0)),
            ],
            out_specs=pl.BlockSpec((NPAD, D), lambda e, f, sb, nb: (0, 0)),
        ),
        out_shape=jax.ShapeDtypeStruct((NPAD, D), jnp.float32),
    )(sb, nb, xs, W1, b1.reshape(E, 1, DFF), W2, b2.reshape(E, 1, D))


def _sc_gather_body(ys_hbm, p0_hbm, p1_hbm, yk0_hbm, yk1_hbm,
                    idx0_v, idx1_v, buf0_v, buf1_v, sem0, sem1):
    wid = lax.axis_index("s") * NC + lax.axis_index("c")
    base = wid * CH
    ci0 = pltpu.async_copy(p0_hbm.at[pl.ds(base, CH)], idx0_v, sem0)
    ci1 = pltpu.async_copy(p1_hbm.at[pl.ds(base, CH)], idx1_v, sem1)
    ci0.wait()
    ci1.wait()
    cp0 = pltpu.async_copy(ys_hbm.at[idx0_v], buf0_v, sem0)
    cp1 = pltpu.async_copy(ys_hbm.at[idx1_v], buf1_v, sem1)
    cp0.wait()
    co0 = pltpu.async_copy(buf0_v, yk0_hbm.at[pl.ds(base, CH), :], sem0)
    cp1.wait()
    co1 = pltpu.async_copy(buf1_v, yk1_hbm.at[pl.ds(base, CH), :], sem1)
    co0.wait()
    co1.wait()


def _sc_gather(ys, p0, p1):
    mesh = plsc.VectorSubcoreMesh(core_axis_name="c", subcore_axis_name="s",
                                  num_cores=NC, num_subcores=NS)
    return pl.kernel(
        _sc_gather_body,
        out_type=[jax.ShapeDtypeStruct((T, D), jnp.float32),
                  jax.ShapeDtypeStruct((T, D), jnp.float32)],
        mesh=mesh,
        scratch_types=[pltpu.VMEM((CH,), jnp.int32),
                       pltpu.VMEM((CH,), jnp.int32),
                       pltpu.VMEM((CH, D), jnp.float32),
                       pltpu.VMEM((CH, D), jnp.float32),
                       pltpu.SemaphoreType.DMA,
                       pltpu.SemaphoreType.DMA],
    )(ys, p0, p1)


def _finish_body(x_ref, yk0_ref, yk1_ref, w0_ref, w1_ref,
                 Ws1_ref, bs1_ref, Ws2_ref, bs2_ref, scale_ref, out_ref):
    xb = x_ref[...]
    hs = jnp.maximum(
        jnp.dot(xb, Ws1_ref[...], preferred_element_type=jnp.float32)
        + bs1_ref[...], 0.0)
    sh = jnp.dot(hs, Ws2_ref[...], preferred_element_type=jnp.float32) + bs2_ref[...]
    s = scale_ref[0, 0]
    moe = w0_ref[...] * yk0_ref[...] + w1_ref[...] * yk1_ref[...]
    out_ref[...] = xb + moe + (xb + sh) * s


def _finish(xf, yk0, yk1, w0, w1, Ws1, bs1, Ws2, bs2, scale):
    return pl.pallas_call(
        _finish_body,
        grid=(NT,),
        in_specs=[pl.BlockSpec((BT, D), lambda t: (t, 0)),
                  pl.BlockSpec((BT, D), lambda t: (t, 0)),
                  pl.BlockSpec((BT, D), lambda t: (t, 0)),
                  pl.BlockSpec((BT, 1), lambda t: (t, 0)),
                  pl.BlockSpec((BT, 1), lambda t: (t, 0)),
                  pl.BlockSpec((D, DFF), lambda t: (0, 0)),
                  pl.BlockSpec((1, DFF), lambda t: (0, 0)),
                  pl.BlockSpec((DFF, D), lambda t: (0, 0)),
                  pl.BlockSpec((1, D), lambda t: (0, 0)),
                  pl.BlockSpec((1, 1), lambda t: (0, 0))],
        out_specs=pl.BlockSpec((BT, D), lambda t: (t, 0)),
        out_shape=jax.ShapeDtypeStruct((T, D), jnp.float32),
    )(xf, yk0, yk1, w0, w1, Ws1, bs1.reshape(1, DFF), Ws2,
      bs2.reshape(1, D), scale.reshape(1, 1))


def kernel(x, Wg, bg, W1, b1, W2, b2, Ws1, bs1, Ws2, bs2, scale):
    xf = x.reshape(T, D)
    p0, p1, w0, w1, sb, nb = _route(xf, Wg, bg)
    p0f, p1f = p0.reshape(T), p1.reshape(T)
    xs = _sc_scatter(xf, p0f, p1f)
    ys = _gmm(sb.reshape(E), nb.reshape(E), xs, W1, b1, W2, b2)
    yk0, yk1 = _sc_gather(ys, p0f, p1f)
    out = _finish(xf, yk0, yk1, w0, w1, Ws1, bs1, Ws2, bs2, scale)
    return out.reshape(x.shape)


# speedup vs baseline: 1.0537x; 1.0066x over previous
import jax
import jax.numpy as jnp
from jax import lax
from jax.experimental import pallas as pl
from jax.experimental.pallas import tpu as pltpu
from jax.experimental.pallas import tpu_sc as plsc

B, S, D, DFF, E, K = 1, 2048, 768, 3072, 8, 2
T = B * S
BTM = 256
NB = (T * K) // BTM + E
NPAD = NB * BTM

NC, NS = 2, 16
NW = NC * NS
CH = T // NW

BT = 512
NT = T // BT


def _csum0(a, n):
    k = 1
    while k < n:
        z = jnp.zeros((k,) + a.shape[1:], a.dtype)
        a = a + jnp.concatenate([z, a[:n - k]], axis=0)
        k *= 2
    return a


def _csum1(a, n):
    k = 1
    while k < n:
        z = jnp.zeros(a.shape[:1] + (k,), a.dtype)
        a = a + jnp.concatenate([z, a[:, :n - k]], axis=1)
        k *= 2
    return a


def _route_body(x_ref, Wg_ref, bg_ref, p0_ref, p1_ref, w0_ref, w1_ref,
                sb_ref, nb_ref):
    logits = jnp.dot(x_ref[...], Wg_ref[...],
                     preferred_element_type=jnp.float32) + bg_ref[...]
    p = jax.nn.softmax(logits, axis=-1)
    ids = lax.broadcasted_iota(jnp.int32, p.shape, 1)
    v1 = jnp.max(p, axis=-1, keepdims=True)
    i1 = jnp.min(jnp.where(p >= v1, ids, E), axis=-1, keepdims=True)
    m1 = ids == i1
    pm = jnp.where(m1, -jnp.inf, p)
    v2 = jnp.max(pm, axis=-1, keepdims=True)
    i2 = jnp.min(jnp.where(pm >= v2, ids, E), axis=-1, keepdims=True)
    m2 = ids == i2
    denom = v1 + v2
    w0_ref[...] = v1 / denom
    w1_ref[...] = v2 / denom

    oh0 = m1.astype(jnp.int32)
    oh1 = m2.astype(jnp.int32)
    c0 = _csum0(oh0, T)
    c1 = _csum0(oh1, T)
    tot0 = c0[T - 1:T, :]
    tot1 = c1[T - 1:T, :]
    tote = tot0 + tot1
    pb = (tote + BTM - 1) // BTM
    sblk = _csum1(pb, E) - pb
    spad = sblk * BTM
    p0_ref[...] = jnp.sum(oh0 * (spad + c0 - 1), axis=1, keepdims=True)
    p1_ref[...] = jnp.sum(oh1 * (spad + tot0 + c1 - 1), axis=1, keepdims=True)
    sb_ref[...] = sblk
    nb_ref[...] = pb


def _route(xf, Wg, bg):
    return pl.pallas_call(
        _route_body,
        in_specs=[pl.BlockSpec((T, D), lambda: (0, 0)),
                  pl.BlockSpec((D, E), lambda: (0, 0)),
                  pl.BlockSpec((1, E), lambda: (0, 0))],
        out_specs=[pl.BlockSpec((T, 1), lambda: (0, 0)),
                   pl.BlockSpec((T, 1), lambda: (0, 0)),
                   pl.BlockSpec((T, 1), lambda: (0, 0)),
                   pl.BlockSpec((T, 1), lambda: (0, 0)),
                   pl.BlockSpec((1, E), lambda: (0, 0)),
                   pl.BlockSpec((1, E), lambda: (0, 0))],
        out_shape=[jax.ShapeDtypeStruct((T, 1), jnp.int32),
                   jax.ShapeDtypeStruct((T, 1), jnp.int32),
                   jax.ShapeDtypeStruct((T, 1), jnp.float32),
                   jax.ShapeDtypeStruct((T, 1), jnp.float32),
                   jax.ShapeDtypeStruct((1, E), jnp.int32),
                   jax.ShapeDtypeStruct((1, E), jnp.int32)],
    )(xf, Wg, bg.reshape(1, E))


def _sc_scatter_body(x_hbm, p0_hbm, p1_hbm, xs_hbm, idx0_v, idx1_v, rows_v,
                     sem0, sem1, semr):
    wid = lax.axis_index("s") * NC + lax.axis_index("c")
    base = wid * CH
    cp0 = pltpu.async_copy(p0_hbm.at[pl.ds(base, CH)], idx0_v, sem0)
    cp1 = pltpu.async_copy(p1_hbm.at[pl.ds(base, CH)], idx1_v, sem1)
    cpr = pltpu.async_copy(x_hbm.at[pl.ds(base, CH), :], rows_v, semr)
    cp0.wait()
    cp1.wait()
    cpr.wait()
    s0 = pltpu.async_copy(rows_v, xs_hbm.at[idx0_v], sem0)
    s1 = pltpu.async_copy(rows_v, xs_hbm.at[idx1_v], sem1)
    s0.wait()
    s1.wait()


def _sc_scatter(xf, p0, p1):
    mesh = plsc.VectorSubcoreMesh(core_axis_name="c", subcore_axis_name="s",
                                  num_cores=NC, num_subcores=NS)
    return pl.kernel(
        _sc_scatter_body,
        out_type=jax.ShapeDtypeStruct((NPAD, D), jnp.float32),
        mesh=mesh,
        scratch_types=[pltpu.VMEM((CH,), jnp.int32),
                       pltpu.VMEM((CH,), jnp.int32),
                       pltpu.VMEM((CH, D), jnp.float32),
                       pltpu.SemaphoreType.DMA,
                       pltpu.SemaphoreType.DMA,
                       pltpu.SemaphoreType.DMA],
    )(xf, p0, p1)


F = 2
DF = DFF // F


def _gmm_body(sb_ref, nb_ref, xs_ref, W1_ref, b1_ref, W2_ref, b2_ref, ys_ref):
    e = pl.program_id(0)
    f = pl.program_id(1)
    base = sb_ref[e] * BTM

    def step(j, carry):
        off = pl.multiple_of(base + j * BTM, BTM)
        xb = xs_ref[pl.ds(off, BTM), :]
        h = jnp.maximum(
            jnp.dot(xb, W1_ref[0], preferred_element_type=jnp.float32)
            + b1_ref[0], 0.0)
        y = jnp.dot(h, W2_ref[0], preferred_element_type=jnp.float32)
        ys_ref[pl.ds(off, BTM), :] = jnp.where(
            f == 0, y + b2_ref[0], ys_ref[pl.ds(off, BTM), :] + y)
        return carry

    lax.fori_loop(0, nb_ref[e], step, 0)


def _gmm(sb, nb, xs, W1, b1, W2, b2):
    return pl.pallas_call(
        _gmm_body,
        grid_spec=pltpu.PrefetchScalarGridSpec(
            num_scalar_prefetch=2,
            grid=(E, F),
            in_specs=[
                pl.BlockSpec((NPAD, D), lambda e, f, sb, nb: (0, 0)),
                pl.BlockSpec((1, D, DF), lambda e, f, sb, nb: (e, 0, f)),
                pl.BlockSpec((1, 1, DF), lambda e, f, sb, nb: (e, 0, f)),
                pl.BlockSpec((1, DF, D), lambda e, f, sb, nb: (e, f, 0)),
                pl.BlockSpec((1, 1, D), lambda e, f, sb, nb: (e, 0, 0)),
            ],
            out_specs=pl.BlockSpec((NPAD, D), lambda e, f, sb, nb: (0, 0)),
        ),
        out_shape=jax.ShapeDtypeStruct((NPAD, D), jnp.float32),
    )(sb, nb, xs, W1, b1.reshape(E, 1, DFF), W2, b2.reshape(E, 1, D))


def _sc_gather_body(ys_hbm, p0_hbm, p1_hbm, yk0_hbm, yk1_hbm,
                    idx0_v, idx1_v, buf0_v, buf1_v, sem0, sem1):
    wid = lax.axis_index("s") * NC + lax.axis_index("c")
    base = wid * CH
    ci0 = pltpu.async_copy(p0_hbm.at[pl.ds(base, CH)], idx0_v, sem0)
    ci1 = pltpu.async_copy(p1_hbm.at[pl.ds(base, CH)], idx1_v, sem1)
    ci0.wait()
    ci1.wait()
    cp0 = pltpu.async_copy(ys_hbm.at[idx0_v], buf0_v, sem0)
    cp1 = pltpu.async_copy(ys_hbm.at[idx1_v], buf1_v, sem1)
    cp0.wait()
    co0 = pltpu.async_copy(buf0_v, yk0_hbm.at[pl.ds(base, CH), :], sem0)
    cp1.wait()
    co1 = pltpu.async_copy(buf1_v, yk1_hbm.at[pl.ds(base, CH), :], sem1)
    co0.wait()
    co1.wait()


def _sc_gather(ys, p0, p1):
    mesh = plsc.VectorSubcoreMesh(core_axis_name="c", subcore_axis_name="s",
                                  num_cores=NC, num_subcores=NS)
    return pl.kernel(
        _sc_gather_body,
        out_type=[jax.ShapeDtypeStruct((T, D), jnp.float32),
                  jax.ShapeDtypeStruct((T, D), jnp.float32)],
        mesh=mesh,
        scratch_types=[pltpu.VMEM((CH,), jnp.int32),
                       pltpu.VMEM((CH,), jnp.int32),
                       pltpu.VMEM((CH, D), jnp.float32),
                       pltpu.VMEM((CH, D), jnp.float32),
                       pltpu.SemaphoreType.DMA,
                       pltpu.SemaphoreType.DMA],
    )(ys, p0, p1)


def _finish_body(x_ref, yk0_ref, yk1_ref, w0_ref, w1_ref,
                 Ws1_ref, bs1_ref, Ws2_ref, bs2_ref, scale_ref, out_ref):
    xb = x_ref[...]
    hs = jnp.maximum(
        jnp.dot(xb, Ws1_ref[...], preferred_element_type=jnp.float32)
        + bs1_ref[...], 0.0)
    sh = jnp.dot(hs, Ws2_ref[...], preferred_element_type=jnp.float32) + bs2_ref[...]
    s = scale_ref[0, 0]
    moe = w0_ref[...] * yk0_ref[...] + w1_ref[...] * yk1_ref[...]
    out_ref[...] = xb + moe + (xb + sh) * s


def _finish(xf, yk0, yk1, w0, w1, Ws1, bs1, Ws2, bs2, scale):
    return pl.pallas_call(
        _finish_body,
        grid=(NT,),
        in_specs=[pl.BlockSpec((BT, D), lambda t: (t, 0)),
                  pl.BlockSpec((BT, D), lambda t: (t, 0)),
                  pl.BlockSpec((BT, D), lambda t: (t, 0)),
                  pl.BlockSpec((BT, 1), lambda t: (t, 0)),
                  pl.BlockSpec((BT, 1), lambda t: (t, 0)),
                  pl.BlockSpec((D, DFF), lambda t: (0, 0)),
                  pl.BlockSpec((1, DFF), lambda t: (0, 0)),
                  pl.BlockSpec((DFF, D), lambda t: (0, 0)),
                  pl.BlockSpec((1, D), lambda t: (0, 0)),
                  pl.BlockSpec((1, 1), lambda t: (0, 0))],
        out_specs=pl.BlockSpec((BT, D), lambda t: (t, 0)),
        out_shape=jax.ShapeDtypeStruct((T, D), jnp.float32),
    )(xf, yk0, yk1, w0, w1, Ws1, bs1.reshape(1, DFF), Ws2,
      bs2.reshape(1, D), scale.reshape(1, 1))


def kernel(x, Wg, bg, W1, b1, W2, b2, Ws1, bs1, Ws2, bs2, scale):
    xf = x.reshape(T, D)
    p0, p1, w0, w1, sb, nb = _route(xf, Wg, bg)
    p0f, p1f = p0.reshape(T), p1.reshape(T)
    xs = _sc_scatter(xf, p0f, p1f)
    ys = _gmm(sb.reshape(E), nb.reshape(E), xs, W1, b1, W2, b2)
    yk0, yk1 = _sc_gather(ys, p0f, p1f)
    out = _finish(xf, yk0, yk1, w0, w1, Ws1, bs1, Ws2, bs2, scale)
    return out.reshape(x.shape)
